# Initial kernel scaffold; baseline (speedup 1.0000x reference)
import functools
import jax
import jax.numpy as jnp
from jax import lax
from jax.experimental import pallas as pl
from jax.experimental.pallas import tpu as pltpu
from jax.experimental.pallas import tpu_sc as plsc

N = 10000
E = 320000
DIN = 14
H = 128
G = 256

NC = 2
NS = 16
NW = NC * NS
NP = 10240
EPW = E // NW
CHUNK = 80
NSTEP = EPW // CHUNK
ZROWS = NP // NS


def _make_edge_segsum(feat):
    mesh = plsc.VectorSubcoreMesh(
        core_axis_name="c", subcore_axis_name="s", num_cores=NC,
        num_subcores=NS)

    @functools.partial(
        pl.kernel,
        out_type=jax.ShapeDtypeStruct((NC, NP, feat), jnp.float32),
        mesh=mesh,
        scratch_types=[
            pltpu.VMEM((CHUNK,), jnp.int32),
            pltpu.VMEM((CHUNK,), jnp.int32),
            pltpu.VMEM((CHUNK, feat), jnp.float32),
            pltpu.VMEM_SHARED((NP, feat), jnp.float32),
            pltpu.SemaphoreType.DMA,
        ],
    )
    def seg_kernel(src_hbm, dst_hbm, vals_hbm, out_hbm, sidx, didx, rows,
                   acc, sem):
        c = lax.axis_index("c")
        s = lax.axis_index("s")
        wid = s * NC + c

        zero = jnp.zeros((16,), jnp.float32)

        def zfill(r, carry):
            for cc in range(feat // 16):
                rows[r, pl.ds(cc * 16, 16)] = zero
            return carry

        lax.fori_loop(0, CHUNK, zfill, 0)
        for k in range(ZROWS // CHUNK):
            pltpu.sync_copy(rows, acc.at[pl.ds(s * ZROWS + k * CHUNK, CHUNK)])
        plsc.subcore_barrier()

        base = wid * EPW

        def step(i, carry):
            off = base + i * CHUNK
            pltpu.sync_copy(src_hbm.at[pl.ds(off, CHUNK)], sidx)
            pltpu.sync_copy(dst_hbm.at[pl.ds(off, CHUNK)], didx)
            pltpu.async_copy(vals_hbm.at[sidx], rows, sem).wait()
            pltpu.sync_copy(rows, acc.at[didx], add=True)
            return carry

        lax.fori_loop(0, NSTEP, step, 0)
        plsc.subcore_barrier()

        pltpu.sync_copy(acc.at[pl.ds(s * ZROWS, ZROWS)],
                        out_hbm.at[c, pl.ds(s * ZROWS, ZROWS)])

    return seg_kernel


_segsum16 = _make_edge_segsum(16)
_segsum128 = _make_edge_segsum(H)

BN = 1024
NBLK = NP // BN


def _mlp1_body(x_ref, p_ref, w1a_ref, b1a_ref, w1b_ref, b1b_ref, out_ref):
    h = x_ref[...] + p_ref[0] + p_ref[1]
    a = jnp.maximum(
        jnp.dot(h, w1a_ref[...], preferred_element_type=jnp.float32)
        + b1a_ref[...], 0.0)
    o = jnp.maximum(
        jnp.dot(a, w1b_ref[...], preferred_element_type=jnp.float32)
        + b1b_ref[...], 0.0)
    out_ref[...] = o


def _mlp2_pool_body(h_ref, q_ref, batch_ref, w2a_ref, b2a_ref, w2b_ref,
                    b2b_ref, wout_ref, bout_ref, out_ref, sums_ref, cnt_ref):
    i = pl.program_id(0)

    @pl.when(i == 0)
    def _():
        sums_ref[...] = jnp.zeros_like(sums_ref)
        cnt_ref[...] = jnp.zeros_like(cnt_ref)

    h = h_ref[...] + q_ref[0] + q_ref[1]
    a = jnp.maximum(
        jnp.dot(h, w2a_ref[...], preferred_element_type=jnp.float32)
        + b2a_ref[...], 0.0)
    h2 = jnp.maximum(
        jnp.dot(a, w2b_ref[...], preferred_element_type=jnp.float32)
        + b2b_ref[...], 0.0)

    b = batch_ref[0, 0, :]
    mask = (b[:, None] == lax.broadcasted_iota(jnp.int32, (BN, G), 1)
            ).astype(jnp.float32)
    sums_ref[...] += lax.dot_general(
        mask, h2, (((0,), (0,)), ((), ())),
        preferred_element_type=jnp.float32)
    cnt_ref[...] += lax.dot_general(
        mask, jnp.ones_like(h2), (((0,), (0,)), ((), ())),
        preferred_element_type=jnp.float32)

    @pl.when(i == NBLK - 1)
    def _():
        pooled = sums_ref[...] / jnp.maximum(cnt_ref[...], 1.0)
        out_ref[...] = jnp.dot(
            pooled, wout_ref[...],
            preferred_element_type=jnp.float32) + bout_ref[...]


def kernel(x, edge_index, batch, W1a, b1a, W1b, b1b, W2a, b2a, W2b, b2b,
           Wout, bout):
    f32 = jnp.float32
    src = edge_index[0]
    dst = edge_index[1]

    x16 = jnp.zeros((NP, 16), f32).at[:N, :DIN].set(x)
    w1a16 = jnp.concatenate([W1a, jnp.zeros((16 - DIN, H), f32)], axis=0)
    batch_pad = jnp.concatenate(
        [batch, jnp.full((NP - N,), G, jnp.int32)]).reshape(NBLK, 1, BN)

    p16 = _segsum16(src, dst, x16)

    h1 = pl.pallas_call(
        _mlp1_body,
        grid=(NBLK,),
        in_specs=[
            pl.BlockSpec((BN, 16), lambda i: (i, 0)),
            pl.BlockSpec((NC, BN, 16), lambda i: (0, i, 0)),
            pl.BlockSpec((16, H), lambda i: (0, 0)),
            pl.BlockSpec((1, H), lambda i: (0, 0)),
            pl.BlockSpec((H, H), lambda i: (0, 0)),
            pl.BlockSpec((1, H), lambda i: (0, 0)),
        ],
        out_specs=pl.BlockSpec((BN, H), lambda i: (i, 0)),
        out_shape=jax.ShapeDtypeStruct((NP, H), f32),
    )(x16, p16, w1a16, b1a.reshape(1, H), W1b, b1b.reshape(1, H))

    q = _segsum128(src, dst, h1)

    out = pl.pallas_call(
        _mlp2_pool_body,
        grid=(NBLK,),
        in_specs=[
            pl.BlockSpec((BN, H), lambda i: (i, 0)),
            pl.BlockSpec((NC, BN, H), lambda i: (0, i, 0)),
            pl.BlockSpec((1, 1, BN), lambda i: (i, 0, 0)),
            pl.BlockSpec((H, H), lambda i: (0, 0)),
            pl.BlockSpec((1, H), lambda i: (0, 0)),
            pl.BlockSpec((H, H), lambda i: (0, 0)),
            pl.BlockSpec((1, H), lambda i: (0, 0)),
            pl.BlockSpec((H, H), lambda i: (0, 0)),
            pl.BlockSpec((1, H), lambda i: (0, 0)),
        ],
        out_specs=pl.BlockSpec((G, H), lambda i: (0, 0)),
        out_shape=jax.ShapeDtypeStruct((G, H), f32),
        scratch_shapes=[
            pltpu.VMEM((G, H), f32),
            pltpu.VMEM((G, H), f32),
        ],
    )(h1, q, batch_pad, W2a, b2a.reshape(1, H), W2b, b2b.reshape(1, H),
      Wout, bout.reshape(1, H))

    return out


# speedup vs baseline: 5.4327x; 5.4327x over previous
import functools
import jax
import jax.numpy as jnp
from jax import lax
from jax.experimental import pallas as pl
from jax.experimental.pallas import tpu as pltpu
from jax.experimental.pallas import tpu_sc as plsc

N = 10000
E = 320000
DIN = 14
H = 128
G = 256

NC = 2
NS = 16
NW = NC * NS
NP = 10240
EPW = E // NW
CHUNK = 80
NSTEP = EPW // CHUNK
ZROWS = NP // NS


def _make_edge_segsum(feat):
    mesh = plsc.VectorSubcoreMesh(
        core_axis_name="c", subcore_axis_name="s", num_cores=NC,
        num_subcores=NS)

    @functools.partial(
        pl.kernel,
        out_type=jax.ShapeDtypeStruct((NC, NP, feat), jnp.float32),
        mesh=mesh,
        compiler_params=pltpu.CompilerParams(use_tc_tiling_on_sc=False),
        scratch_types=[
            pltpu.VMEM((CHUNK,), jnp.int32),
            pltpu.VMEM((CHUNK,), jnp.int32),
            pltpu.VMEM((CHUNK, feat), jnp.float32),
            pltpu.VMEM_SHARED((NP, feat), jnp.float32),
            pltpu.SemaphoreType.DMA,
        ],
    )
    def seg_kernel(src_hbm, dst_hbm, vals_hbm, out_hbm, sidx, didx, rows,
                   acc, sem):
        c = lax.axis_index("c")
        s = lax.axis_index("s")
        wid = s * NC + c

        zero = jnp.zeros((16,), jnp.float32)

        def zfill(r, carry):
            for cc in range(feat // 16):
                rows[r, pl.ds(cc * 16, 16)] = zero
            return carry

        lax.fori_loop(0, CHUNK, zfill, 0)
        for k in range(ZROWS // CHUNK):
            pltpu.sync_copy(rows, acc.at[pl.ds(s * ZROWS + k * CHUNK, CHUNK)])
        plsc.subcore_barrier()

        base = wid * EPW

        def step(i, carry):
            off = base + i * CHUNK
            pltpu.sync_copy(src_hbm.at[pl.ds(off, CHUNK)], sidx)
            pltpu.sync_copy(dst_hbm.at[pl.ds(off, CHUNK)], didx)
            pltpu.async_copy(vals_hbm.at[sidx], rows, sem).wait()
            pltpu.sync_copy(rows, acc.at[didx], add=True)
            return carry

        lax.fori_loop(0, NSTEP, step, 0)
        plsc.subcore_barrier()

        pltpu.sync_copy(acc.at[pl.ds(s * ZROWS, ZROWS)],
                        out_hbm.at[c, pl.ds(s * ZROWS, ZROWS)])

    return seg_kernel


_segsum16 = _make_edge_segsum(16)
_segsum128 = _make_edge_segsum(H)

BN = 1024
NBLK = NP // BN


def _mlp1_body(x_ref, p_ref, w1a_ref, b1a_ref, w1b_ref, b1b_ref, out_ref):
    h = x_ref[...] + p_ref[0] + p_ref[1]
    a = jnp.maximum(
        jnp.dot(h, w1a_ref[...], preferred_element_type=jnp.float32)
        + b1a_ref[...], 0.0)
    o = jnp.maximum(
        jnp.dot(a, w1b_ref[...], preferred_element_type=jnp.float32)
        + b1b_ref[...], 0.0)
    out_ref[...] = o


def _mlp2_pool_body(h_ref, q_ref, batch_ref, w2a_ref, b2a_ref, w2b_ref,
                    b2b_ref, wout_ref, bout_ref, out_ref, sums_ref, cnt_ref):
    i = pl.program_id(0)

    @pl.when(i == 0)
    def _():
        sums_ref[...] = jnp.zeros_like(sums_ref)
        cnt_ref[...] = jnp.zeros_like(cnt_ref)

    h = h_ref[...] + q_ref[0] + q_ref[1]
    a = jnp.maximum(
        jnp.dot(h, w2a_ref[...], preferred_element_type=jnp.float32)
        + b2a_ref[...], 0.0)
    h2 = jnp.maximum(
        jnp.dot(a, w2b_ref[...], preferred_element_type=jnp.float32)
        + b2b_ref[...], 0.0)

    b = batch_ref[0, 0, :]
    mask = (b[:, None] == lax.broadcasted_iota(jnp.int32, (BN, G), 1)
            ).astype(jnp.float32)
    sums_ref[...] += lax.dot_general(
        mask, h2, (((0,), (0,)), ((), ())),
        preferred_element_type=jnp.float32)
    cnt_ref[...] += lax.dot_general(
        mask, jnp.ones_like(h2), (((0,), (0,)), ((), ())),
        preferred_element_type=jnp.float32)

    @pl.when(i == NBLK - 1)
    def _():
        pooled = sums_ref[...] / jnp.maximum(cnt_ref[...], 1.0)
        out_ref[...] = jnp.dot(
            pooled, wout_ref[...],
            preferred_element_type=jnp.float32) + bout_ref[...]


def kernel(x, edge_index, batch, W1a, b1a, W1b, b1b, W2a, b2a, W2b, b2b,
           Wout, bout):
    f32 = jnp.float32
    src = edge_index[0]
    dst = edge_index[1]

    x16 = jnp.zeros((NP, 16), f32).at[:N, :DIN].set(x)
    w1a16 = jnp.concatenate([W1a, jnp.zeros((16 - DIN, H), f32)], axis=0)
    batch_pad = jnp.concatenate(
        [batch, jnp.full((NP - N,), G, jnp.int32)]).reshape(NBLK, 1, BN)

    p16 = _segsum16(src, dst, x16)

    h1 = pl.pallas_call(
        _mlp1_body,
        grid=(NBLK,),
        in_specs=[
            pl.BlockSpec((BN, 16), lambda i: (i, 0)),
            pl.BlockSpec((NC, BN, 16), lambda i: (0, i, 0)),
            pl.BlockSpec((16, H), lambda i: (0, 0)),
            pl.BlockSpec((1, H), lambda i: (0, 0)),
            pl.BlockSpec((H, H), lambda i: (0, 0)),
            pl.BlockSpec((1, H), lambda i: (0, 0)),
        ],
        out_specs=pl.BlockSpec((BN, H), lambda i: (i, 0)),
        out_shape=jax.ShapeDtypeStruct((NP, H), f32),
    )(x16, p16, w1a16, b1a.reshape(1, H), W1b, b1b.reshape(1, H))

    q = _segsum128(src, dst, h1)

    out = pl.pallas_call(
        _mlp2_pool_body,
        grid=(NBLK,),
        in_specs=[
            pl.BlockSpec((BN, H), lambda i: (i, 0)),
            pl.BlockSpec((NC, BN, H), lambda i: (0, i, 0)),
            pl.BlockSpec((1, 1, BN), lambda i: (i, 0, 0)),
            pl.BlockSpec((H, H), lambda i: (0, 0)),
            pl.BlockSpec((1, H), lambda i: (0, 0)),
            pl.BlockSpec((H, H), lambda i: (0, 0)),
            pl.BlockSpec((1, H), lambda i: (0, 0)),
            pl.BlockSpec((H, H), lambda i: (0, 0)),
            pl.BlockSpec((1, H), lambda i: (0, 0)),
        ],
        out_specs=pl.BlockSpec((G, H), lambda i: (0, 0)),
        out_shape=jax.ShapeDtypeStruct((G, H), f32),
        scratch_shapes=[
            pltpu.VMEM((G, H), f32),
            pltpu.VMEM((G, H), f32),
        ],
    )(h1, q, batch_pad, W2a, b2a.reshape(1, H), W2b, b2b.reshape(1, H),
      Wout, bout.reshape(1, H))

    return out


# speedup vs baseline: 15.5456x; 2.8615x over previous
import functools
import jax
import jax.numpy as jnp
from jax import lax
from jax.experimental import pallas as pl
from jax.experimental.pallas import tpu as pltpu
from jax.experimental.pallas import tpu_sc as plsc

N = 10000
E = 320000
DIN = 14
H = 128
G = 256

NC = 2
NS = 16
NW = NC * NS
NP = 10240
EPW = E // NW
CHUNK = 80
NSTEP = EPW // CHUNK
ZROWS = NP // NS


NB = 3


def _make_edge_segsum(feat):
    mesh = plsc.VectorSubcoreMesh(
        core_axis_name="c", subcore_axis_name="s", num_cores=NC,
        num_subcores=NS)

    @functools.partial(
        pl.kernel,
        out_type=jax.ShapeDtypeStruct((NC, NP, feat), jnp.float32),
        mesh=mesh,
        compiler_params=pltpu.CompilerParams(use_tc_tiling_on_sc=False),
        scratch_types=[
            pltpu.VMEM((NSTEP, CHUNK), jnp.int32),
            [pltpu.VMEM((CHUNK,), jnp.int32) for _ in range(NB)],
            [pltpu.VMEM((CHUNK, feat), jnp.float32) for _ in range(NB)],
            pltpu.VMEM_SHARED((NP, feat), jnp.float32),
            [pltpu.SemaphoreType.DMA for _ in range(NB)],
            [pltpu.SemaphoreType.DMA for _ in range(NB)],
        ],
    )
    def seg_kernel(src_hbm, dst_hbm, vals_hbm, out_hbm, sidx, didx, rows,
                   acc, gsems, dsems):
        c = lax.axis_index("c")
        s = lax.axis_index("s")
        wid = s * NC + c

        zero = jnp.zeros((16,), jnp.float32)

        def zfill(r, carry):
            for cc in range(feat // 16):
                rows[0][r, pl.ds(cc * 16, 16)] = zero
            return carry

        lax.fori_loop(0, CHUNK, zfill, 0)
        for k in range(ZROWS // CHUNK):
            pltpu.sync_copy(rows[0],
                            acc.at[pl.ds(s * ZROWS + k * CHUNK, CHUNK)])

        pltpu.sync_copy(src_hbm.at[wid], sidx)
        plsc.subcore_barrier()

        def issue(i, b):
            pltpu.async_copy(dst_hbm.at[wid, i], didx[b], dsems[b])
            pltpu.async_copy(vals_hbm.at[sidx.at[i]], rows[b], gsems[b])

        def drain(i, b):
            pltpu.make_async_copy(
                dst_hbm.at[wid, i], didx[b], dsems[b]).wait()
            pltpu.make_async_copy(
                vals_hbm.at[sidx.at[i]], rows[b], gsems[b]).wait()
            pltpu.sync_copy(rows[b], acc.at[didx[b]], add=True)

        for b in range(NB):
            issue(b, b)

        def step(j, carry):
            for b in range(NB):
                i = j * NB + b
                drain(i, b)

                @pl.when(i + NB < NSTEP)
                def _():
                    issue(i + NB, b)
            return carry

        lax.fori_loop(0, NSTEP // NB, step, 0)
        for b in range(NSTEP % NB):
            drain(NSTEP - (NSTEP % NB) + b, b)

        plsc.subcore_barrier()
        pltpu.sync_copy(acc.at[pl.ds(s * ZROWS, ZROWS)],
                        out_hbm.at[c, pl.ds(s * ZROWS, ZROWS)])

    return seg_kernel


_segsum16 = _make_edge_segsum(16)
_segsum128 = _make_edge_segsum(H)

BN = 1024
NBLK = NP // BN


def _mlp1_body(x_ref, p_ref, w1a_ref, b1a_ref, w1b_ref, b1b_ref, out_ref):
    h = x_ref[...] + p_ref[0] + p_ref[1]
    a = jnp.maximum(
        jnp.dot(h, w1a_ref[...], preferred_element_type=jnp.float32)
        + b1a_ref[...], 0.0)
    o = jnp.maximum(
        jnp.dot(a, w1b_ref[...], preferred_element_type=jnp.float32)
        + b1b_ref[...], 0.0)
    out_ref[...] = o


def _mlp2_pool_body(h_ref, q_ref, batch_ref, w2a_ref, b2a_ref, w2b_ref,
                    b2b_ref, wout_ref, bout_ref, out_ref, sums_ref, cnt_ref):
    i = pl.program_id(0)

    @pl.when(i == 0)
    def _():
        sums_ref[...] = jnp.zeros_like(sums_ref)
        cnt_ref[...] = jnp.zeros_like(cnt_ref)

    h = h_ref[...] + q_ref[0] + q_ref[1]
    a = jnp.maximum(
        jnp.dot(h, w2a_ref[...], preferred_element_type=jnp.float32)
        + b2a_ref[...], 0.0)
    h2 = jnp.maximum(
        jnp.dot(a, w2b_ref[...], preferred_element_type=jnp.float32)
        + b2b_ref[...], 0.0)

    b = batch_ref[0, 0, :]
    mask = (b[:, None] == lax.broadcasted_iota(jnp.int32, (BN, G), 1)
            ).astype(jnp.float32)
    sums_ref[...] += lax.dot_general(
        mask, h2, (((0,), (0,)), ((), ())),
        preferred_element_type=jnp.float32)
    cnt_ref[...] += lax.dot_general(
        mask, jnp.ones_like(h2), (((0,), (0,)), ((), ())),
        preferred_element_type=jnp.float32)

    @pl.when(i == NBLK - 1)
    def _():
        pooled = sums_ref[...] / jnp.maximum(cnt_ref[...], 1.0)
        out_ref[...] = jnp.dot(
            pooled, wout_ref[...],
            preferred_element_type=jnp.float32) + bout_ref[...]


def kernel(x, edge_index, batch, W1a, b1a, W1b, b1b, W2a, b2a, W2b, b2b,
           Wout, bout):
    f32 = jnp.float32
    src = edge_index[0].reshape(NW, NSTEP, CHUNK)
    dst = edge_index[1].reshape(NW, NSTEP, CHUNK)

    x16 = jnp.zeros((NP, 16), f32).at[:N, :DIN].set(x)
    w1a16 = jnp.concatenate([W1a, jnp.zeros((16 - DIN, H), f32)], axis=0)
    batch_pad = jnp.concatenate(
        [batch, jnp.full((NP - N,), G, jnp.int32)]).reshape(NBLK, 1, BN)

    p16 = _segsum16(src, dst, x16)

    h1 = pl.pallas_call(
        _mlp1_body,
        grid=(NBLK,),
        in_specs=[
            pl.BlockSpec((BN, 16), lambda i: (i, 0)),
            pl.BlockSpec((NC, BN, 16), lambda i: (0, i, 0)),
            pl.BlockSpec((16, H), lambda i: (0, 0)),
            pl.BlockSpec((1, H), lambda i: (0, 0)),
            pl.BlockSpec((H, H), lambda i: (0, 0)),
            pl.BlockSpec((1, H), lambda i: (0, 0)),
        ],
        out_specs=pl.BlockSpec((BN, H), lambda i: (i, 0)),
        out_shape=jax.ShapeDtypeStruct((NP, H), f32),
    )(x16, p16, w1a16, b1a.reshape(1, H), W1b, b1b.reshape(1, H))

    q = _segsum128(src, dst, h1)

    out = pl.pallas_call(
        _mlp2_pool_body,
        grid=(NBLK,),
        in_specs=[
            pl.BlockSpec((BN, H), lambda i: (i, 0)),
            pl.BlockSpec((NC, BN, H), lambda i: (0, i, 0)),
            pl.BlockSpec((1, 1, BN), lambda i: (i, 0, 0)),
            pl.BlockSpec((H, H), lambda i: (0, 0)),
            pl.BlockSpec((1, H), lambda i: (0, 0)),
            pl.BlockSpec((H, H), lambda i: (0, 0)),
            pl.BlockSpec((1, H), lambda i: (0, 0)),
            pl.BlockSpec((H, H), lambda i: (0, 0)),
            pl.BlockSpec((1, H), lambda i: (0, 0)),
        ],
        out_specs=pl.BlockSpec((G, H), lambda i: (0, 0)),
        out_shape=jax.ShapeDtypeStruct((G, H), f32),
        scratch_shapes=[
            pltpu.VMEM((G, H), f32),
            pltpu.VMEM((G, H), f32),
        ],
    )(h1, q, batch_pad, W2a, b2a.reshape(1, H), W2b, b2b.reshape(1, H),
      Wout, bout.reshape(1, H))

    return out


# speedup vs baseline: 17.4003x; 1.1193x over previous
import functools
import jax
import jax.numpy as jnp
from jax import lax
from jax.experimental import pallas as pl
from jax.experimental.pallas import tpu as pltpu
from jax.experimental.pallas import tpu_sc as plsc

N = 10000
E = 320000
DIN = 14
H = 128
G = 256

NC = 2
NS = 16
NW = NC * NS
NP = 10240
EPW = E // NW
ZROWS = NP // NS


def _make_edge_segsum(feat, chunk, nb):
    nstep = EPW // chunk
    assert EPW % chunk == 0 and chunk % 8 == 0 and chunk <= 128
    mesh = plsc.VectorSubcoreMesh(
        core_axis_name="c", subcore_axis_name="s", num_cores=NC,
        num_subcores=NS)

    @functools.partial(
        pl.kernel,
        out_type=jax.ShapeDtypeStruct((NC, NP, feat), jnp.float32),
        mesh=mesh,
        compiler_params=pltpu.CompilerParams(use_tc_tiling_on_sc=False),
        scratch_types=[
            pltpu.VMEM((EPW,), jnp.int32),
            [pltpu.VMEM((chunk,), jnp.int32) for _ in range(nb)],
            [pltpu.VMEM((chunk, feat), jnp.float32) for _ in range(nb)],
            pltpu.VMEM_SHARED((NP, feat), jnp.float32),
            [pltpu.SemaphoreType.DMA for _ in range(nb)],
            [pltpu.SemaphoreType.DMA for _ in range(nb)],
        ],
    )
    def seg_kernel(edge_hbm, vals_hbm, out_hbm, sidx, didx, rows, acc,
                   gsems, dsems):
        c = lax.axis_index("c")
        s = lax.axis_index("s")
        wid = s * NC + c
        base = wid * EPW

        @pl.when(c == 0)
        def _():
            pltpu.sync_copy(vals_hbm.at[pl.ds(s * ZROWS, ZROWS)],
                            acc.at[pl.ds(s * ZROWS, ZROWS)])

        @pl.when(c != 0)
        def _():
            zero = jnp.zeros((16,), jnp.float32)

            def zfill(r, carry):
                for cc in range(feat // 16):
                    rows[0][r, pl.ds(cc * 16, 16)] = zero
                return carry

            lax.fori_loop(0, chunk, zfill, 0)
            for k in range(ZROWS // chunk):
                pltpu.sync_copy(
                    rows[0], acc.at[pl.ds(s * ZROWS + k * chunk, chunk)])

        pltpu.sync_copy(edge_hbm.at[0, pl.ds(base, EPW)], sidx)
        plsc.subcore_barrier()

        def issue(i, b):
            pltpu.async_copy(edge_hbm.at[1, pl.ds(base + i * chunk, chunk)],
                             didx[b], dsems[b])
            pltpu.async_copy(vals_hbm.at[sidx.at[pl.ds(i * chunk, chunk)]],
                             rows[b], gsems[b])

        def drain(i, b):
            pltpu.make_async_copy(
                edge_hbm.at[1, pl.ds(base + i * chunk, chunk)],
                didx[b], dsems[b]).wait()
            pltpu.make_async_copy(
                vals_hbm.at[sidx.at[pl.ds(i * chunk, chunk)]],
                rows[b], gsems[b]).wait()
            pltpu.sync_copy(rows[b], acc.at[didx[b]], add=True)

        for b in range(nb):
            issue(b, b)

        def step(j, carry):
            for b in range(nb):
                i = j * nb + b
                drain(i, b)

                @pl.when(i + nb < nstep)
                def _():
                    issue(i + nb, b)
            return carry

        lax.fori_loop(0, nstep // nb, step, 0)
        for b in range(nstep % nb):
            drain(nstep - (nstep % nb) + b, b)

        plsc.subcore_barrier()
        pltpu.sync_copy(acc.at[pl.ds(s * ZROWS, ZROWS)],
                        out_hbm.at[c, pl.ds(s * ZROWS, ZROWS)])

    return seg_kernel


_segsum16 = _make_edge_segsum(16, 80, 5)
_segsum128 = _make_edge_segsum(H, 80, 3)

BN = 1024
NBLK = NP // BN


def _mlp1_body(p_ref, w1a_ref, b1a_ref, w1b_ref, b1b_ref, out_ref):
    h = p_ref[0] + p_ref[1]
    a = jnp.maximum(
        jnp.dot(h, w1a_ref[...], preferred_element_type=jnp.float32)
        + b1a_ref[...], 0.0)
    o = jnp.maximum(
        jnp.dot(a, w1b_ref[...], preferred_element_type=jnp.float32)
        + b1b_ref[...], 0.0)
    out_ref[...] = o


def _mlp2_pool_body(q_ref, batch_ref, w2a_ref, b2a_ref, w2b_ref,
                    b2b_ref, wout_ref, bout_ref, out_ref, sums_ref, cnt_ref):
    i = pl.program_id(0)

    @pl.when(i == 0)
    def _():
        sums_ref[...] = jnp.zeros_like(sums_ref)
        cnt_ref[...] = jnp.zeros_like(cnt_ref)

    h = q_ref[0] + q_ref[1]
    a = jnp.maximum(
        jnp.dot(h, w2a_ref[...], preferred_element_type=jnp.float32)
        + b2a_ref[...], 0.0)
    h2 = jnp.maximum(
        jnp.dot(a, w2b_ref[...], preferred_element_type=jnp.float32)
        + b2b_ref[...], 0.0)

    b = batch_ref[0, 0, :]
    mask = (b[:, None] == lax.broadcasted_iota(jnp.int32, (BN, G), 1)
            ).astype(jnp.float32)
    sums_ref[...] += lax.dot_general(
        mask, h2, (((0,), (0,)), ((), ())),
        preferred_element_type=jnp.float32)
    cnt_ref[...] += lax.dot_general(
        mask, jnp.ones_like(h2), (((0,), (0,)), ((), ())),
        preferred_element_type=jnp.float32)

    @pl.when(i == NBLK - 1)
    def _():
        pooled = sums_ref[...] / jnp.maximum(cnt_ref[...], 1.0)
        out_ref[...] = jnp.dot(
            pooled, wout_ref[...],
            preferred_element_type=jnp.float32) + bout_ref[...]


def kernel(x, edge_index, batch, W1a, b1a, W1b, b1b, W2a, b2a, W2b, b2b,
           Wout, bout):
    f32 = jnp.float32

    x16 = jnp.zeros((NP, 16), f32).at[:N, :DIN].set(x)
    w1a16 = jnp.concatenate([W1a, jnp.zeros((16 - DIN, H), f32)], axis=0)
    batch_pad = jnp.concatenate(
        [batch, jnp.full((NP - N,), G, jnp.int32)]).reshape(NBLK, 1, BN)

    p16 = _segsum16(edge_index, x16)

    h1 = pl.pallas_call(
        _mlp1_body,
        grid=(NBLK,),
        in_specs=[
            pl.BlockSpec((NC, BN, 16), lambda i: (0, i, 0)),
            pl.BlockSpec((16, H), lambda i: (0, 0)),
            pl.BlockSpec((1, H), lambda i: (0, 0)),
            pl.BlockSpec((H, H), lambda i: (0, 0)),
            pl.BlockSpec((1, H), lambda i: (0, 0)),
        ],
        out_specs=pl.BlockSpec((BN, H), lambda i: (i, 0)),
        out_shape=jax.ShapeDtypeStruct((NP, H), f32),
    )(p16, w1a16, b1a.reshape(1, H), W1b, b1b.reshape(1, H))

    q = _segsum128(edge_index, h1)

    out = pl.pallas_call(
        _mlp2_pool_body,
        grid=(NBLK,),
        in_specs=[
            pl.BlockSpec((NC, BN, H), lambda i: (0, i, 0)),
            pl.BlockSpec((1, 1, BN), lambda i: (i, 0, 0)),
            pl.BlockSpec((H, H), lambda i: (0, 0)),
            pl.BlockSpec((1, H), lambda i: (0, 0)),
            pl.BlockSpec((H, H), lambda i: (0, 0)),
            pl.BlockSpec((1, H), lambda i: (0, 0)),
            pl.BlockSpec((H, H), lambda i: (0, 0)),
            pl.BlockSpec((1, H), lambda i: (0, 0)),
        ],
        out_specs=pl.BlockSpec((G, H), lambda i: (0, 0)),
        out_shape=jax.ShapeDtypeStruct((G, H), f32),
        scratch_shapes=[
            pltpu.VMEM((G, H), f32),
            pltpu.VMEM((G, H), f32),
        ],
    )(q, batch_pad, W2a, b2a.reshape(1, H), W2b, b2b.reshape(1, H),
      Wout, bout.reshape(1, H))

    return out
